# Initial kernel scaffold; baseline (speedup 1.0000x reference)
import jax
import jax.numpy as jnp
from jax.experimental import pallas as pl


def kernel(output, targets):
    raise NotImplementedError("write your pallas kernel here")


# speedup vs baseline: 35.9899x; 35.9899x over previous
import jax
import jax.numpy as jnp
import numpy as np
from jax.experimental import pallas as pl
from jax.experimental.pallas import tpu as pltpu

_ANCHORS = np.array([[1.08, 1.19], [3.42, 4.41], [6.63, 11.38],
                     [9.42, 5.11], [16.62, 10.52]], dtype=np.float32)
_NC = 8
_NA = 5
_CH = 7 + _NC
_THR = 0.6
_B, _H, _W, _MO = 16, 64, 64, 20
_HW = _H * _W


def _body(out_ref, tgt_b_ref, tgt_all_ref, loss_ref, g_ref, acc_ref):
    b = pl.program_id(0)
    out = out_ref[0]

    conf = jnp.concatenate(
        [out[_CH * k + 6:_CH * k + 7, :] for k in range(_NA)], axis=0)
    p = jax.nn.sigmoid(conf)
    dsum = jnp.sum(jnp.minimum(-jnp.log(1.0 - p), 100.0))

    @pl.when(b == 0)
    def _():
        acc_ref[0] = 0.0
    acc_ref[0] = acc_ref[0] + dsum

    tg_b = tgt_b_ref[0]
    axb = (tg_b[:, 1:2] * float(_H)).astype(jnp.int32)
    ayb = (tg_b[:, 2:3] * float(_W)).astype(jnp.int32)
    pix = axb * _W + ayb
    colidx = jax.lax.broadcasted_iota(jnp.int32, (_MO, _HW), 1)
    P = (colidx == pix).astype(jnp.float32)
    G = jax.lax.dot_general(P, out, (((1,), (1,)), ((), ())),
                            preferred_element_type=jnp.float32,
                            precision=jax.lax.Precision.HIGHEST)
    g_ref[pl.ds(b, 1)] = G.reshape(1, _MO, _NA * _CH)

    @pl.when(b == _B - 1)
    def _():
        tg = tgt_all_ref[:, :, :]
        Gall = g_ref[:, :, :]

        cls = tg[:, :, 0]
        notf = jnp.where(cls == 0.0, 1.0, 0.0)
        row_i = jax.lax.broadcasted_iota(jnp.int32, (_MO, _MO), 0)
        col_j = jax.lax.broadcasted_iota(jnp.int32, (_MO, _MO), 1)
        lower = col_j <= row_i
        bad3 = jnp.where(lower[None], notf[:, None, :], 0.0)
        active = jnp.max(bad3, axis=2) == 0.0

        gt_x = tg[:, :, 1] * float(_H)
        gt_y = tg[:, :, 2] * float(_W)
        gt_l = tg[:, :, 3] * float(_H)
        gt_w = tg[:, :, 4] * float(_W)
        gim = tg[:, :, 5]
        gre = tg[:, :, 6]

        k5 = jax.lax.broadcasted_iota(jnp.int32, (1, 1, _NA), 2)
        bl = jnp.zeros((1, 1, _NA), jnp.float32)
        bw = jnp.zeros((1, 1, _NA), jnp.float32)
        for k in range(_NA):
            bl = jnp.where(k5 == k, float(_ANCHORS[k, 0]), bl)
            bw = jnp.where(k5 == k, float(_ANCHORS[k, 1]), bw)
        gl3 = gt_l[:, :, None]
        gw3 = gt_w[:, :, None]
        min_x = jnp.minimum(0.0 - gl3 / 2.0, 0.0 - bl / 2.0)
        max_x = jnp.maximum(0.0 + gl3 / 2.0, 0.0 + bl / 2.0)
        min_y = jnp.minimum(0.0 - gw3 / 2.0, 0.0 - bw / 2.0)
        max_y = jnp.maximum(0.0 + gw3 / 2.0, 0.0 + bw / 2.0)
        union_w = max_y - min_y
        union_h = max_x - min_x
        inter_w = gw3 + bw - union_w
        inter_l = gl3 + bl - union_h
        badi = (inter_w <= 0) | (inter_l <= 0)
        inter_areas = jnp.where(badi, 0.0, inter_w * inter_l)
        union_areas = gw3 * gl3 + bw * bl - inter_areas
        ious = inter_areas / union_areas

        iou_max = jnp.max(ious, axis=2, keepdims=True)
        kidx = jax.lax.broadcasted_iota(jnp.int32, (_B, _MO, _NA), 2)
        a = jnp.min(jnp.where(ious == iou_max, kidx, 99), axis=2)

        ax = gt_x.astype(jnp.int32)
        ay = gt_y.astype(jnp.int32)
        fx = gt_x - ax.astype(jnp.float32)
        fy = gt_y - ay.astype(jnp.float32)

        al_sel = jnp.zeros((_B, _MO), jnp.float32)
        aw_sel = jnp.zeros((_B, _MO), jnp.float32)
        for k in range(_NA):
            mk = a == k
            al_sel = jnp.where(mk, float(_ANCHORS[k, 0]), al_sel)
            aw_sel = jnp.where(mk, float(_ANCHORS[k, 1]), aw_sel)
        safe_gl = jnp.where(active, gt_l, 1.0)
        safe_gw = jnp.where(active, gt_w, 1.0)
        tl = jnp.log(safe_gl / al_sel)
        tw = jnp.log(safe_gw / aw_sel)

        act2 = active[:, :, None] & active[:, None, :]
        same_col = ((ax[:, :, None] == ax[:, None, :])
                    & (ay[:, :, None] == ay[:, None, :]) & act2)
        same_cell = same_col & (a[:, :, None] == a[:, None, :])
        jgt = (col_j > row_i)[None]
        last_cell = active & ~jnp.any(same_cell & jgt, axis=2)
        last_col = active & ~jnp.any(same_col & jgt, axis=2)
        cnt_t = jnp.sum(last_cell.astype(jnp.float32))

        cls_id = cls.astype(jnp.int32)
        label = jnp.min(jnp.where(same_cell, cls_id[:, None, :], 9999),
                        axis=2)

        sel = jnp.zeros((_B, _MO, _CH), jnp.float32)
        for k in range(_NA):
            sel = jnp.where((a == k)[:, :, None],
                            Gall[:, :, _CH * k:_CH * (k + 1)], sel)
        conf_all = jnp.concatenate(
            [Gall[:, :, _CH * k + 6:_CH * k + 7] for k in range(_NA)],
            axis=2)

        o_x = jax.nn.sigmoid(sel[:, :, 0])
        o_y = jax.nn.sigmoid(sel[:, :, 1])
        o_l = sel[:, :, 2]
        o_w = sel[:, :, 3]
        o_im = sel[:, :, 4]
        o_re = sel[:, :, 5]
        conf_p = jax.nn.sigmoid(sel[:, :, 6])
        cls_logit = sel[:, :, 7:]

        sq = ((o_x - fx) ** 2 + (o_y - fy) ** 2 + (o_l - tl) ** 2
              + (o_w - tw) ** 2 + (o_im - gim) ** 2 + (o_re - gre) ** 2)
        conf_true = -jnp.maximum(jnp.log(conf_p), -100.0)
        num_t = jnp.sum(jnp.where(last_cell, sq + conf_true, 0.0))

        pc = jax.nn.sigmoid(cls_logit)
        mx = jnp.max(pc, axis=2, keepdims=True)
        sh = pc - mx
        logp = sh - jnp.log(jnp.sum(jnp.exp(sh), axis=2, keepdims=True))
        cidx = jax.lax.broadcasted_iota(jnp.int32, (_B, _MO, _NC), 2)
        picked = jnp.sum(jnp.where(cidx == label[:, :, None], logp, 0.0),
                         axis=2)
        num_cls = jnp.sum(jnp.where(last_cell, -picked, 0.0))

        corr_conf = jnp.float32(0.0)
        corr_cnt = jnp.float32(0.0)
        for k in range(_NA):
            a_eq = a == k
            hi = ious[:, :, k] > _THR
            evt = a_eq | hi
            exists = same_col & evt[:, None, :]
            jl = jnp.max(jnp.where(exists, col_j[None], -1), axis=2)
            lastsel = exists & (col_j[None] == jl[:, :, None])
            cm = (jnp.sum(jnp.where(lastsel,
                                    a_eq.astype(jnp.float32)[:, None, :],
                                    0.0), axis=2)
                  + (jl < 0).astype(jnp.float32))
            tm = jnp.any(same_col & a_eq[:, None, :],
                         axis=2).astype(jnp.float32)
            fm = cm - tm
            x = conf_all[:, :, k]
            px = jax.nn.sigmoid(x)
            lp = jnp.maximum(jnp.log(px), -100.0)
            l1p = jnp.maximum(jnp.log(1.0 - px), -100.0)
            assumed = -l1p
            actual = fm * (-(tm * lp + (1.0 - tm) * l1p))
            corr_conf += jnp.sum(jnp.where(last_col, actual - assumed, 0.0))
            corr_cnt += jnp.sum(jnp.where(last_col, fm - 1.0, 0.0))

        cnt_f = float(_B * _NA * _HW) + corr_cnt
        dense_num = acc_ref[0] + corr_conf

        loss = (num_t / cnt_t + dense_num / cnt_f
                + num_cls / (float(_B) * cnt_t))
        loss_ref[:, :] = jnp.full((1, 1), loss, jnp.float32)


def kernel(output, targets):
    out3 = output.reshape(_B, _NA * _CH, _HW)
    loss = pl.pallas_call(
        _body,
        grid=(_B,),
        in_specs=[
            pl.BlockSpec((1, _NA * _CH, _HW), lambda b: (b, 0, 0)),
            pl.BlockSpec((1, _MO, 7), lambda b: (b, 0, 0)),
            pl.BlockSpec((_B, _MO, 7), lambda b: (0, 0, 0)),
        ],
        out_specs=pl.BlockSpec((1, 1), lambda b: (0, 0)),
        out_shape=jax.ShapeDtypeStruct((1, 1), jnp.float32),
        scratch_shapes=[
            pltpu.VMEM((_B, _MO, _NA * _CH), jnp.float32),
            pltpu.SMEM((1,), jnp.float32),
        ],
    )(out3, targets, targets)
    return loss[0, 0]
